# baseline (speedup 1.0000x reference)
import functools

import jax
import jax.numpy as jnp
from jax import lax
from jax.experimental import pallas as pl
from jax.experimental.pallas import tpu as pltpu
from jax.experimental.pallas import tpu_sc as plsc


def _mm_multi(xs, w, b, douts, relu_add, bn=2048):
    n, k = xs[0].shape
    dtot = w.shape[1]
    assert sum(douts) == dtot
    grid = (pl.cdiv(n, bn),)
    nx = len(xs)

    def body(*refs):
        xrefs = refs[:nx]
        w_ref = refs[nx]
        b_ref = refs[nx + 1]
        orefs = refs[nx + 2:]
        x = xrefs[0][...]
        if nx == 2:
            x = x + xrefs[1][...]
        if relu_add:
            x = jnp.maximum(x, 0.0)
        y = jnp.dot(x, w_ref[...], preferred_element_type=jnp.float32)
        y = y + b_ref[...]
        off = 0
        for o_ref, dd in zip(orefs, douts):
            o_ref[...] = y[:, off:off + dd]
            off += dd

    in_specs = [pl.BlockSpec((bn, k), lambda i: (i, 0)) for _ in xs]
    in_specs.append(pl.BlockSpec((k, dtot), lambda i: (0, 0)))
    in_specs.append(pl.BlockSpec((1, dtot), lambda i: (0, 0)))
    out_specs = [pl.BlockSpec((bn, dd), lambda i: (i, 0)) for dd in douts]
    out_shape = [jax.ShapeDtypeStruct((n, dd), jnp.float32) for dd in douts]
    outs = pl.pallas_call(
        body,
        grid=grid,
        in_specs=in_specs,
        out_specs=out_specs,
        out_shape=out_shape,
    )(*xs, w, b[None])
    return outs


_EB = 2048
_DP = 128
_CH = 8192
_G = 64


def _gat_edges(xl, xr, src, dst, att, bias, n_dst, d_real):
    ch = _CH if d_real >= 128 else 2 * _CH
    info = plsc.get_sparse_core_info()
    nc, ns = info.num_cores, info.num_subcores
    nch = pl.cdiv(n_dst, ch)
    n_out = nch * ch
    e_pad = src.shape[0]
    assert e_pad % (ns * _EB) == 0
    ept = e_pad // ns
    nblk = ept // _EB
    zr = 32
    rows_per_tile = ch // ns
    assert rows_per_tile % zr == 0
    dp = d_real
    nb = 64
    nr = d_real // 16
    dstride = ch + 128
    ebg = _EB // _G + 1

    mesh = plsc.VectorSubcoreMesh(core_axis_name="c", subcore_axis_name="s")

    @functools.partial(
        pl.kernel,
        mesh=mesh,
        compiler_params=pltpu.CompilerParams(needs_layout_passes=False, use_tc_tiling_on_sc=False),
        out_type=jax.ShapeDtypeStruct((n_out, d_real), jnp.float32),
        scratch_types=[
            pltpu.VMEM_SHARED((ch + 16, dp), jnp.float32),
            pltpu.VMEM_SHARED((ns * (ch + 128),), jnp.float32),
            pltpu.VMEM((_EB,), jnp.int32),
            pltpu.VMEM((_EB,), jnp.int32),
            pltpu.VMEM((_EB + 16,), jnp.int32),
            pltpu.VMEM((_EB + 16,), jnp.int32),
            pltpu.VMEM((16, dp), jnp.float32),
            pltpu.VMEM((16, dp), jnp.float32),
            pltpu.VMEM((16, dp), jnp.float32),
            pltpu.VMEM((16, dp), jnp.float32),
            pltpu.VMEM((ch + 128,), jnp.float32),
            pltpu.VMEM((zr, dp), jnp.float32),
            pltpu.VMEM((nb, dp), jnp.float32),
            pltpu.VMEM((16, 16), jnp.float32),
            pltpu.VMEM((ns, nb), jnp.float32),
            pltpu.VMEM((nb,), jnp.float32),
            pltpu.VMEM((d_real,), jnp.float32),
            pltpu.VMEM((d_real,), jnp.float32),
            pltpu.SemaphoreType.DMA,
            pltpu.SemaphoreType.DMA,
            pltpu.SemaphoreType.DMA,
            pltpu.SemaphoreType.DMA,
            pltpu.SemaphoreType.DMA,
            pltpu.SemaphoreType.DMA,
        ],
    )
    def ker(xl_hbm, xr_hbm, src_hbm, dst_hbm, att_hbm, bias_hbm, out_hbm,
            acc, dall, eb_src, eb_dst, csrc, cdrel,
            rows0, rows1, xrr0, xrr1, dacc, zrow,
            nbuf, obuf, dbuf, recbuf, attb, biasb,
            gsem0, gsem1, xsem0, xsem1, ssem0, ssem1):
        cid = lax.axis_index("c")
        sid = lax.axis_index("s")
        pltpu.sync_copy(att_hbm, attb)
        pltpu.sync_copy(bias_hbm, biasb)
        att_vecs = [attb[pl.ds(k * 16, 16)] for k in range(nr)]
        bias_vecs = [biasb[pl.ds(k * 16, 16)] for k in range(nr)]
        iota16 = lax.iota(jnp.int32, 16)
        zvec = jnp.zeros((16,), jnp.float32)

        def zz(i, c):
            for k in range(dp // 16):
                zrow[i, pl.ds(k * 16, 16)] = zvec
            return c

        lax.fori_loop(0, zr, zz, 0)

        nmy = (nch - cid + 1) // 2

        def chunk_body(kk, c0):
            chn = 2 * kk + cid
            lo = chn * ch
            lov = jnp.full((16,), lo, jnp.int32)

            def zslice(z, c):
                pltpu.sync_copy(
                    zrow, acc.at[pl.ds(sid * rows_per_tile + z * zr, zr)])
                return c

            lax.fori_loop(0, rows_per_tile // zr, zslice, 0)

            def zdacc(z, c):
                dacc[pl.ds(z * 16, 16)] = zvec
                return c

            lax.fori_loop(0, dstride // 16, zdacc, 0)
            plsc.subcore_barrier()

            bufs = [(rows0, xrr0, gsem0, xsem0, ssem0),
                    (rows1, xrr1, gsem1, xsem1, ssem1)]

            def issue(j, b):
                rows_b, xrr_b, gsem_b, xsem_b, _ = bufs[b]
                siv = csrc[pl.ds(j * 16, 16)]
                dgv = cdrel[pl.ds(j * 16, 16)] + lov
                pltpu.async_copy(xl_hbm.at[siv], rows_b, gsem_b)
                pltpu.async_copy(xr_hbm.at[dgv], xrr_b, xsem_b)

            def wait_gathers(j, b):
                rows_b, xrr_b, gsem_b, xsem_b, _ = bufs[b]
                siv = csrc[pl.ds(j * 16, 16)]
                dgv = cdrel[pl.ds(j * 16, 16)] + lov
                pltpu.make_async_copy(
                    xl_hbm.at[siv], rows_b, gsem_b).wait()
                pltpu.make_async_copy(
                    xr_hbm.at[dgv], xrr_b, xsem_b).wait()

            def wait_scatter(b):
                rows_b, _, _, _, ssem_b = bufs[b]
                drv0 = cdrel[pl.ds(0, 16)]
                pltpu.make_async_copy(
                    rows_b, acc.at[drv0], ssem_b).wait()

            def compute(j, b):
                rows_b, xrr_b, _, _, ssem_b = bufs[b]
                drv = cdrel[pl.ds(j * 16, 16)]
                for e in range(16):
                    accv = zvec
                    xls = []
                    for k in range(nr):
                        a = rows_b[e, pl.ds(k * 16, 16)]
                        xls.append(a)
                        z = a + xrr_b[e, pl.ds(k * 16, 16)]
                        lr = jnp.maximum(z, 0.2 * z)
                        accv = accv + lr * att_vecs[k]
                    alpha = jnp.sum(accv)
                    pv = jnp.exp(jnp.full((16,), alpha, jnp.float32))
                    for k in range(nr):
                        rows_b[e, pl.ds(k * 16, 16)] = xls[k] * pv
                    plsc.addupdate_scatter(
                        dacc, [drv], pv, mask=(iota16 == e))
                pltpu.async_copy(rows_b, acc.at[drv], ssem_b, add=True)

            def blk_body(bb, c):
                e0 = sid * ept + bb * _EB
                pltpu.sync_copy(src_hbm.at[pl.ds(e0, _EB)], eb_src)
                pltpu.sync_copy(dst_hbm.at[pl.ds(e0, _EB)], eb_dst)

                def cvec(v, cnt):
                    dv = eb_dst[pl.ds(v * 16, 16)]
                    sv = eb_src[pl.ds(v * 16, 16)]
                    dr = dv - lov
                    m = (dr >= 0) & (dr < ch)
                    mi = m.astype(jnp.int32)
                    incl = plsc.cumsum(mi)
                    pos = jnp.full((16,), cnt, jnp.int32) + incl - mi
                    idx = jnp.where(m, pos, _EB)
                    plsc.store_scatter(csrc, [idx], sv)
                    plsc.store_scatter(cdrel, [idx], dr)
                    return cnt + jnp.max(incl)

                cnt = lax.fori_loop(0, _EB // 16, cvec, jnp.int32(0))
                idxp = jnp.full((16,), cnt, jnp.int32) + iota16
                plsc.store_scatter(csrc, [idxp], jnp.zeros((16,), jnp.int32))
                plsc.store_scatter(cdrel, [idxp],
                                   jnp.full((16,), ch, jnp.int32))
                ng = (cnt + 15) // 16

                @pl.when(ng > 0)
                def _():
                    issue(0, 0)

                def g2(jj, c2):
                    for b in range(2):
                        j = jj * 2 + b

                        @pl.when(j < ng)
                        def _(j=j, b=b):
                            @pl.when(j + 1 < ng)
                            def _():
                                @pl.when(j >= 1)
                                def _():
                                    wait_scatter(1 - b)

                                issue(j + 1, 1 - b)

                            wait_gathers(j, b)
                            compute(j, b)
                    return c2

                lax.fori_loop(0, (ng + 1) // 2, g2, 0)

                @pl.when(ng >= 2)
                def _():
                    wait_scatter(0)
                    wait_scatter(1)

                @pl.when(ng == 1)
                def _():
                    wait_scatter(0)

                return c

            lax.fori_loop(0, nblk, blk_body, 0)

            pltpu.sync_copy(dacc, dall.at[pl.ds(sid * dstride, dstride)])
            plsc.subcore_barrier()

            r0 = sid * rows_per_tile

            def norm(g, c):
                gb = r0 + g * nb
                pltpu.sync_copy(acc.at[pl.ds(gb, nb)], nbuf)
                for t in range(ns):
                    pltpu.sync_copy(
                        dall.at[pl.ds(t * dstride + gb, nb)], dbuf.at[t])
                for rb in range(nb // 16):
                    densv = zvec
                    for t in range(ns):
                        densv = densv + dbuf[t, pl.ds(rb * 16, 16)]
                    recbuf[pl.ds(rb * 16, 16)] = 1.0 / (densv + 1e-16)

                tgt = nbuf

                def nrow(r, c2):
                    rv = plsc.load_gather(recbuf, [jnp.full((16,), r, jnp.int32)])
                    for k in range(nr):
                        tgt[r, pl.ds(k * 16, 16)] = (
                            nbuf[r, pl.ds(k * 16, 16)] * rv + bias_vecs[k])
                    return c2

                lax.fori_loop(0, nb, nrow, 0)
                pltpu.sync_copy(tgt, out_hbm.at[pl.ds(lo + gb, nb)])
                return c

            lax.fori_loop(0, rows_per_tile // nb, norm, 0)
            plsc.subcore_barrier()
            return c0

        lax.fori_loop(0, nmy, chunk_body, 0)

    out = ker(xl, xr, src, dst, att, bias)
    return out[:n_dst]


def _cpad(a, w=_DP):
    if a.ndim == 1:
        return jnp.pad(a, (0, w - a.shape[0]))
    return jnp.pad(a, ((0, 0), (0, w - a.shape[1])))


def _pad_edges(ei, ns_eb=16 * _EB):
    e = ei.shape[1]
    e_pad = ((e + ns_eb - 1) // ns_eb) * ns_eb
    src = jnp.pad(ei[0], (0, e_pad - e), constant_values=0)
    dst = jnp.pad(ei[1], (0, e_pad - e), constant_values=-1)
    return src, dst


def kernel(x_news, x_entities, x_topic, ei_news_has_entities,
           ei_entities_in_news, ei_news_on_topic, ei_topic_in_news,
           ei_entities_similar_entities, params):
    l1, l2 = params["layers"]
    lin = params["lin"]
    n_news = x_news.shape[0]
    n_ent = x_entities.shape[0]
    n_top = x_topic.shape[0]

    s_has, d_has = _pad_edges(ei_news_has_entities)
    s_ein, d_ein = _pad_edges(ei_entities_in_news)
    s_on, d_on = _pad_edges(ei_news_on_topic)
    s_tin, d_tin = _pad_edges(ei_topic_in_news)
    s_sim, d_sim = _pad_edges(ei_entities_similar_entities)

    wn = jnp.concatenate(
        [l1["has"]["Wl"], l1["on"]["Wl"], l1["ein"]["Wr"], l1["tin"]["Wr"]],
        axis=1)
    bn_ = jnp.concatenate(
        [l1["has"]["bl"], l1["on"]["bl"], l1["ein"]["br"], l1["tin"]["br"]])
    xl_has, xl_on, xr_ein, xr_tin = _mm_multi(
        [x_news], wn, bn_, [128, 128, 128, 128], False)

    we = jnp.concatenate(
        [l1["ein"]["Wl"], l1["sim"]["Wl"], l1["has"]["Wr"], l1["sim"]["Wr"]],
        axis=1)
    be_ = jnp.concatenate(
        [l1["ein"]["bl"], l1["sim"]["bl"], l1["has"]["br"], l1["sim"]["br"]])
    xl_ein, xl_sim, xr_has, xr_sim = _mm_multi(
        [x_entities], we, be_, [128, 128, 128, 128], False)

    wt = jnp.concatenate([l1["tin"]["Wl"], l1["on"]["Wr"]], axis=1)
    bt_ = jnp.concatenate([l1["tin"]["bl"], l1["on"]["br"]])
    xl_tin, xr_on = _mm_multi([x_topic], wt, bt_, [128, 128], False)

    gat_has = _gat_edges(xl_has, xr_has, s_has, d_has, l1["has"]["att"],
                         l1["has"]["bias"], n_ent, 128)
    gat_sim = _gat_edges(xl_sim, xr_sim, s_sim, d_sim, l1["sim"]["att"],
                         l1["sim"]["bias"], n_ent, 128)
    gat_ein = _gat_edges(xl_ein, xr_ein, s_ein, d_ein, l1["ein"]["att"],
                         l1["ein"]["bias"], n_news, 128)
    gat_tin = _gat_edges(xl_tin, xr_tin, s_tin, d_tin, l1["tin"]["att"],
                         l1["tin"]["bias"], n_news, 128)
    gat_on = _gat_edges(xl_on, xr_on, s_on, d_on, l1["on"]["att"],
                        l1["on"]["bias"], n_top, 128)

    wn2 = jnp.concatenate([l2["ein"]["Wr"], l2["tin"]["Wr"]], axis=1)
    bn2 = jnp.concatenate([l2["ein"]["br"], l2["tin"]["br"]])
    xr2_ein, xr2_tin = _mm_multi(
        [gat_ein, gat_tin], wn2, bn2, [64, 64], True)

    (xl2_ein,) = _mm_multi(
        [gat_has, gat_sim], l2["ein"]["Wl"], l2["ein"]["bl"], [64], True)
    (xl2_tin,) = _mm_multi(
        [gat_on], l2["tin"]["Wl"], l2["tin"]["bl"], [64], True)

    gat2_ein = _gat_edges(xl2_ein, xr2_ein, s_ein, d_ein, l2["ein"]["att"],
                          l2["ein"]["bias"], n_news, 64)
    gat2_tin = _gat_edges(xl2_tin, xr2_tin, s_tin, d_tin, l2["tin"]["att"],
                          l2["tin"]["bias"], n_news, 64)

    (out,) = _mm_multi([gat2_ein, gat2_tin], lin["W"], lin["b"], [64], True)
    return out


# speedup vs baseline: 1.0065x; 1.0065x over previous
import functools

import jax
import jax.numpy as jnp
from jax import lax
from jax.experimental import pallas as pl
from jax.experimental.pallas import tpu as pltpu
from jax.experimental.pallas import tpu_sc as plsc


def _mm_multi(xs, w, b, douts, relu_add, bn=2048):
    n, k = xs[0].shape
    dtot = w.shape[1]
    assert sum(douts) == dtot
    grid = (pl.cdiv(n, bn),)
    nx = len(xs)

    def body(*refs):
        xrefs = refs[:nx]
        w_ref = refs[nx]
        b_ref = refs[nx + 1]
        orefs = refs[nx + 2:]
        x = xrefs[0][...]
        if nx == 2:
            x = x + xrefs[1][...]
        if relu_add:
            x = jnp.maximum(x, 0.0)
        y = jnp.dot(x, w_ref[...], preferred_element_type=jnp.float32)
        y = y + b_ref[...]
        off = 0
        for o_ref, dd in zip(orefs, douts):
            o_ref[...] = y[:, off:off + dd]
            off += dd

    in_specs = [pl.BlockSpec((bn, k), lambda i: (i, 0)) for _ in xs]
    in_specs.append(pl.BlockSpec((k, dtot), lambda i: (0, 0)))
    in_specs.append(pl.BlockSpec((1, dtot), lambda i: (0, 0)))
    out_specs = [pl.BlockSpec((bn, dd), lambda i: (i, 0)) for dd in douts]
    out_shape = [jax.ShapeDtypeStruct((n, dd), jnp.float32) for dd in douts]
    outs = pl.pallas_call(
        body,
        grid=grid,
        in_specs=in_specs,
        out_specs=out_specs,
        out_shape=out_shape,
    )(*xs, w, b[None])
    return outs


_EB = 2048
_DP = 128
_CH = 8192
_G = 64


def _gat_edges(xl, xr, src, dst, att, bias, n_dst, d_real):
    ch = _CH
    info = plsc.get_sparse_core_info()
    nc, ns = info.num_cores, info.num_subcores
    nch = pl.cdiv(n_dst, ch)
    n_out = nch * ch
    e_pad = src.shape[0]
    assert e_pad % (ns * _EB) == 0
    ept = e_pad // ns
    nblk = ept // _EB
    zr = 32
    rows_per_tile = ch // ns
    assert rows_per_tile % zr == 0
    dp = d_real
    nb = 64
    nr = d_real // 16
    dstride = ch + 128
    ebg = _EB // _G + 1

    mesh = plsc.VectorSubcoreMesh(core_axis_name="c", subcore_axis_name="s")

    @functools.partial(
        pl.kernel,
        mesh=mesh,
        compiler_params=pltpu.CompilerParams(needs_layout_passes=False, use_tc_tiling_on_sc=False),
        out_type=jax.ShapeDtypeStruct((n_out, d_real), jnp.float32),
        scratch_types=[
            pltpu.VMEM_SHARED((ch + 16, dp), jnp.float32),
            pltpu.VMEM_SHARED((ns * (ch + 128),), jnp.float32),
            pltpu.VMEM((_EB,), jnp.int32),
            pltpu.VMEM((_EB,), jnp.int32),
            pltpu.VMEM((_EB + 16,), jnp.int32),
            pltpu.VMEM((_EB + 16,), jnp.int32),
            pltpu.VMEM((16, dp), jnp.float32),
            pltpu.VMEM((16, dp), jnp.float32),
            pltpu.VMEM((16, dp), jnp.float32),
            pltpu.VMEM((16, dp), jnp.float32),
            pltpu.VMEM((ch + 128,), jnp.float32),
            pltpu.VMEM((zr, dp), jnp.float32),
            pltpu.VMEM((nb, dp), jnp.float32),
            pltpu.VMEM((16, 16), jnp.float32),
            pltpu.VMEM((ns, nb), jnp.float32),
            pltpu.VMEM((nb,), jnp.float32),
            pltpu.VMEM((d_real,), jnp.float32),
            pltpu.VMEM((d_real,), jnp.float32),
            pltpu.SemaphoreType.DMA,
            pltpu.SemaphoreType.DMA,
            pltpu.SemaphoreType.DMA,
            pltpu.SemaphoreType.DMA,
            pltpu.SemaphoreType.DMA,
            pltpu.SemaphoreType.DMA,
        ],
    )
    def ker(xl_hbm, xr_hbm, src_hbm, dst_hbm, att_hbm, bias_hbm, out_hbm,
            acc, dall, eb_src, eb_dst, csrc, cdrel,
            rows0, rows1, xrr0, xrr1, dacc, zrow,
            nbuf, obuf, dbuf, recbuf, attb, biasb,
            gsem0, gsem1, xsem0, xsem1, ssem0, ssem1):
        cid = lax.axis_index("c")
        sid = lax.axis_index("s")
        pltpu.sync_copy(att_hbm, attb)
        pltpu.sync_copy(bias_hbm, biasb)
        att_vecs = [attb[pl.ds(k * 16, 16)] for k in range(nr)]
        bias_vecs = [biasb[pl.ds(k * 16, 16)] for k in range(nr)]
        iota16 = lax.iota(jnp.int32, 16)
        zvec = jnp.zeros((16,), jnp.float32)

        def zz(i, c):
            for k in range(dp // 16):
                zrow[i, pl.ds(k * 16, 16)] = zvec
            return c

        lax.fori_loop(0, zr, zz, 0)

        nmy = (nch - cid + 1) // 2

        def chunk_body(kk, c0):
            chn = 2 * kk + cid
            lo = chn * ch
            lov = jnp.full((16,), lo, jnp.int32)

            def zslice(z, c):
                pltpu.sync_copy(
                    zrow, acc.at[pl.ds(sid * rows_per_tile + z * zr, zr)])
                return c

            lax.fori_loop(0, rows_per_tile // zr, zslice, 0)

            def zdacc(z, c):
                dacc[pl.ds(z * 16, 16)] = zvec
                return c

            lax.fori_loop(0, dstride // 16, zdacc, 0)
            plsc.subcore_barrier()

            bufs = [(rows0, xrr0, gsem0, xsem0, ssem0),
                    (rows1, xrr1, gsem1, xsem1, ssem1)]

            def issue(j, b):
                rows_b, xrr_b, gsem_b, xsem_b, _ = bufs[b]
                siv = csrc[pl.ds(j * 16, 16)]
                dgv = cdrel[pl.ds(j * 16, 16)] + lov
                pltpu.async_copy(xl_hbm.at[siv], rows_b, gsem_b)
                pltpu.async_copy(xr_hbm.at[dgv], xrr_b, xsem_b)

            def wait_gathers(j, b):
                rows_b, xrr_b, gsem_b, xsem_b, _ = bufs[b]
                siv = csrc[pl.ds(j * 16, 16)]
                dgv = cdrel[pl.ds(j * 16, 16)] + lov
                pltpu.make_async_copy(
                    xl_hbm.at[siv], rows_b, gsem_b).wait()
                pltpu.make_async_copy(
                    xr_hbm.at[dgv], xrr_b, xsem_b).wait()

            def wait_scatter(b):
                rows_b, _, _, _, ssem_b = bufs[b]
                drv0 = cdrel[pl.ds(0, 16)]
                pltpu.make_async_copy(
                    rows_b, acc.at[drv0], ssem_b).wait()

            def compute(j, b):
                rows_b, xrr_b, _, _, ssem_b = bufs[b]
                drv = cdrel[pl.ds(j * 16, 16)]
                for e in range(16):
                    accv = zvec
                    xls = []
                    for k in range(nr):
                        a = rows_b[e, pl.ds(k * 16, 16)]
                        xls.append(a)
                        z = a + xrr_b[e, pl.ds(k * 16, 16)]
                        lr = jnp.maximum(z, 0.2 * z)
                        accv = accv + lr * att_vecs[k]
                    alpha = jnp.sum(accv)
                    pv = jnp.exp(jnp.full((16,), alpha, jnp.float32))
                    for k in range(nr):
                        rows_b[e, pl.ds(k * 16, 16)] = xls[k] * pv
                    plsc.addupdate_scatter(
                        dacc, [drv], pv, mask=(iota16 == e))
                pltpu.async_copy(rows_b, acc.at[drv], ssem_b, add=True)

            def blk_body(bb, c):
                e0 = sid * ept + bb * _EB
                pltpu.sync_copy(src_hbm.at[pl.ds(e0, _EB)], eb_src)
                pltpu.sync_copy(dst_hbm.at[pl.ds(e0, _EB)], eb_dst)

                def cvec(v, cnt):
                    dv = eb_dst[pl.ds(v * 16, 16)]
                    sv = eb_src[pl.ds(v * 16, 16)]
                    dr = dv - lov
                    m = (dr >= 0) & (dr < ch)
                    mi = m.astype(jnp.int32)
                    incl = plsc.cumsum(mi)
                    pos = jnp.full((16,), cnt, jnp.int32) + incl - mi
                    idx = jnp.where(m, pos, _EB)
                    plsc.store_scatter(csrc, [idx], sv)
                    plsc.store_scatter(cdrel, [idx], dr)
                    return cnt + jnp.max(incl)

                cnt = lax.fori_loop(0, _EB // 16, cvec, jnp.int32(0))
                idxp = jnp.full((16,), cnt, jnp.int32) + iota16
                plsc.store_scatter(csrc, [idxp], jnp.zeros((16,), jnp.int32))
                plsc.store_scatter(cdrel, [idxp],
                                   jnp.full((16,), ch, jnp.int32))
                ng = (cnt + 15) // 16

                @pl.when(ng > 0)
                def _():
                    issue(0, 0)

                def g2(jj, c2):
                    for b in range(2):
                        j = jj * 2 + b

                        @pl.when(j < ng)
                        def _(j=j, b=b):
                            @pl.when(j + 1 < ng)
                            def _():
                                @pl.when(j >= 1)
                                def _():
                                    wait_scatter(1 - b)

                                issue(j + 1, 1 - b)

                            wait_gathers(j, b)
                            compute(j, b)
                    return c2

                lax.fori_loop(0, (ng + 1) // 2, g2, 0)

                @pl.when(ng >= 2)
                def _():
                    wait_scatter(0)
                    wait_scatter(1)

                @pl.when(ng == 1)
                def _():
                    wait_scatter(0)

                return c

            lax.fori_loop(0, nblk, blk_body, 0)

            pltpu.sync_copy(dacc, dall.at[pl.ds(sid * dstride, dstride)])
            plsc.subcore_barrier()

            r0 = sid * rows_per_tile

            def norm(g, c):
                gb = r0 + g * nb
                pltpu.sync_copy(acc.at[pl.ds(gb, nb)], nbuf)
                for t in range(ns):
                    pltpu.sync_copy(
                        dall.at[pl.ds(t * dstride + gb, nb)], dbuf.at[t])
                for rb in range(nb // 16):
                    densv = zvec
                    for t in range(ns):
                        densv = densv + dbuf[t, pl.ds(rb * 16, 16)]
                    recbuf[pl.ds(rb * 16, 16)] = 1.0 / (densv + 1e-16)

                tgt = nbuf

                def nrow(r, c2):
                    rv = plsc.load_gather(recbuf, [jnp.full((16,), r, jnp.int32)])
                    for k in range(nr):
                        tgt[r, pl.ds(k * 16, 16)] = (
                            nbuf[r, pl.ds(k * 16, 16)] * rv + bias_vecs[k])
                    return c2

                lax.fori_loop(0, nb, nrow, 0)
                pltpu.sync_copy(tgt, out_hbm.at[pl.ds(lo + gb, nb)])
                return c

            lax.fori_loop(0, rows_per_tile // nb, norm, 0)
            plsc.subcore_barrier()
            return c0

        lax.fori_loop(0, nmy, chunk_body, 0)

    out = ker(xl, xr, src, dst, att, bias)
    return out[:n_dst]


def _cpad(a, w=_DP):
    if a.ndim == 1:
        return jnp.pad(a, (0, w - a.shape[0]))
    return jnp.pad(a, ((0, 0), (0, w - a.shape[1])))


def _pad_edges(ei, ns_eb=16 * _EB):
    e = ei.shape[1]
    e_pad = ((e + ns_eb - 1) // ns_eb) * ns_eb
    src = jnp.pad(ei[0], (0, e_pad - e), constant_values=0)
    dst = jnp.pad(ei[1], (0, e_pad - e), constant_values=-1)
    return src, dst


def kernel(x_news, x_entities, x_topic, ei_news_has_entities,
           ei_entities_in_news, ei_news_on_topic, ei_topic_in_news,
           ei_entities_similar_entities, params):
    l1, l2 = params["layers"]
    lin = params["lin"]
    n_news = x_news.shape[0]
    n_ent = x_entities.shape[0]
    n_top = x_topic.shape[0]

    s_has, d_has = _pad_edges(ei_news_has_entities)
    s_ein, d_ein = _pad_edges(ei_entities_in_news)
    s_on, d_on = _pad_edges(ei_news_on_topic)
    s_tin, d_tin = _pad_edges(ei_topic_in_news)
    s_sim, d_sim = _pad_edges(ei_entities_similar_entities)

    wn = jnp.concatenate(
        [l1["has"]["Wl"], l1["on"]["Wl"], l1["ein"]["Wr"], l1["tin"]["Wr"]],
        axis=1)
    bn_ = jnp.concatenate(
        [l1["has"]["bl"], l1["on"]["bl"], l1["ein"]["br"], l1["tin"]["br"]])
    xl_has, xl_on, xr_ein, xr_tin = _mm_multi(
        [x_news], wn, bn_, [128, 128, 128, 128], False)

    we = jnp.concatenate(
        [l1["ein"]["Wl"], l1["sim"]["Wl"], l1["has"]["Wr"], l1["sim"]["Wr"]],
        axis=1)
    be_ = jnp.concatenate(
        [l1["ein"]["bl"], l1["sim"]["bl"], l1["has"]["br"], l1["sim"]["br"]])
    xl_ein, xl_sim, xr_has, xr_sim = _mm_multi(
        [x_entities], we, be_, [128, 128, 128, 128], False)

    wt = jnp.concatenate([l1["tin"]["Wl"], l1["on"]["Wr"]], axis=1)
    bt_ = jnp.concatenate([l1["tin"]["bl"], l1["on"]["br"]])
    xl_tin, xr_on = _mm_multi([x_topic], wt, bt_, [128, 128], False)

    gat_has = _gat_edges(xl_has, xr_has, s_has, d_has, l1["has"]["att"],
                         l1["has"]["bias"], n_ent, 128)
    gat_sim = _gat_edges(xl_sim, xr_sim, s_sim, d_sim, l1["sim"]["att"],
                         l1["sim"]["bias"], n_ent, 128)
    gat_ein = _gat_edges(xl_ein, xr_ein, s_ein, d_ein, l1["ein"]["att"],
                         l1["ein"]["bias"], n_news, 128)
    gat_tin = _gat_edges(xl_tin, xr_tin, s_tin, d_tin, l1["tin"]["att"],
                         l1["tin"]["bias"], n_news, 128)
    gat_on = _gat_edges(xl_on, xr_on, s_on, d_on, l1["on"]["att"],
                        l1["on"]["bias"], n_top, 128)

    wn2 = jnp.concatenate([l2["ein"]["Wr"], l2["tin"]["Wr"]], axis=1)
    bn2 = jnp.concatenate([l2["ein"]["br"], l2["tin"]["br"]])
    xr2_ein, xr2_tin = _mm_multi(
        [gat_ein, gat_tin], wn2, bn2, [64, 64], True)

    (xl2_ein,) = _mm_multi(
        [gat_has, gat_sim], l2["ein"]["Wl"], l2["ein"]["bl"], [64], True)
    (xl2_tin,) = _mm_multi(
        [gat_on], l2["tin"]["Wl"], l2["tin"]["bl"], [64], True)

    gat2_ein = _gat_edges(xl2_ein, xr2_ein, s_ein, d_ein, l2["ein"]["att"],
                          l2["ein"]["bias"], n_news, 64)
    gat2_tin = _gat_edges(xl2_tin, xr2_tin, s_tin, d_tin, l2["tin"]["att"],
                          l2["tin"]["bias"], n_news, 64)

    (out,) = _mm_multi([gat2_ein, gat2_tin], lin["W"], lin["b"], [64], True)
    return out
